# baseline (speedup 1.0000x reference)
import jax
import jax.numpy as jnp
from jax import lax
from jax.experimental import pallas as pl
from jax.experimental.pallas import tpu as pltpu
from jax.experimental.pallas import tpu_sc as plsc

NUM_FIELDS = 26
VOCAB = 100000
EMBED_DIM = 32
BATCH = 16384

_info = plsc.get_sparse_core_info()
NC, NS, L = _info.num_cores, _info.num_subcores, _info.num_lanes
NW = NC * NS
Q = 24960
TAIL = VOCAB - 4 * Q
SLAB = 4096
NSLAB = BATCH // SLAB


def _body(xt_hbm, tt_hbm, out_hbm, tv, tvt, xf, ob, sem_t, sem_x, sem_o):
    e = lax.axis_index("s") * NC + lax.axis_index("c")

    def t_copy(f):
        copies = [
            pltpu.make_async_copy(
                tt_hbm.at[f, e, pl.ds(q * Q, Q)], tv.at[pl.ds(q * Q, Q)], sem_t)
            for q in range(4)
        ]
        copies.append(pltpu.make_async_copy(
            tt_hbm.at[f, e, pl.ds(4 * Q, TAIL)], tvt, sem_t))
        return copies

    def x_copy(f, s, par):
        return pltpu.make_async_copy(
            xt_hbm.at[f, pl.ds(s * SLAB, SLAB)], xf.at[par], sem_x)

    def o_copy(f, s):
        return pltpu.make_async_copy(
            ob.at[pl.ds(s * SLAB, SLAB)],
            out_hbm.at[f * EMBED_DIM + e, pl.ds(s * SLAB, SLAB)], sem_o)

    for c in t_copy(0):
        c.start()
    x_copy(0, 0, 0).start()

    def field(f, _):
        for c in t_copy(f):
            c.wait()
        for i in range(TAIL // L):
            tv[pl.ds(4 * Q + i * L, L)] = tvt[pl.ds(i * L, L)]
        for s in range(NSLAB):
            par = s % 2
            x_copy(f, s, par).wait()
            if s + 1 < NSLAB:
                x_copy(f, s + 1, (s + 1) % 2).start()
            else:
                @pl.when(f + 1 < NUM_FIELDS)
                def _():
                    x_copy(f + 1, 0, 0).start()

            @pl.when(f > 0)
            def _():
                o_copy(f - 1, s).wait()

            base = s * SLAB
            for v in range(SLAB // L):
                sl = pl.ds(v * L, L)
                ob[pl.ds(base + v * L, L)] = plsc.load_gather(tv, [xf[par, sl]])
            o_copy(f, s).start()

        @pl.when(f + 1 < NUM_FIELDS)
        def _():
            for c in t_copy(f + 1):
                c.start()
        return 0

    lax.fori_loop(0, NUM_FIELDS, field, 0)
    for s in range(NSLAB):
        o_copy(NUM_FIELDS - 1, s).wait()


@jax.jit
def kernel(x, tables):
    xt = x.T
    tt = jnp.swapaxes(tables, 1, 2)
    fn = pl.kernel(
        _body,
        out_type=jax.ShapeDtypeStruct((NUM_FIELDS * EMBED_DIM, BATCH),
                                      jnp.float32),
        mesh=plsc.VectorSubcoreMesh(core_axis_name="c", subcore_axis_name="s"),
        scratch_types=[
            pltpu.VMEM((VOCAB,), jnp.float32),
            pltpu.VMEM((TAIL,), jnp.float32),
            pltpu.VMEM((2, SLAB), jnp.int32),
            pltpu.VMEM((BATCH,), jnp.float32),
            pltpu.SemaphoreType.DMA,
            pltpu.SemaphoreType.DMA,
            pltpu.SemaphoreType.DMA,
        ],
        compiler_params=pltpu.CompilerParams(needs_layout_passes=False),
    )
    out_t = fn(xt, tt)
    return out_t.T.reshape(BATCH, 1, NUM_FIELDS * EMBED_DIM)
